# baseline (speedup 1.0000x reference)
import functools

import jax
import jax.numpy as jnp
from jax import lax
from jax.experimental import pallas as pl
from jax.experimental.pallas import tpu as pltpu
from jax.experimental.pallas import tpu_sc as plsc

N = 10000
E = 160000
H = 512
G = 64
R = 2000
NT = N // R
_EPS = 1e-5


_EB = 2000
_NBLK = E // _EB
_NPAD = 10240
_MB = 2112


def _splat_to_scalar(v, nbits):
    out = jnp.int32(0)
    for b in range(nbits):
        bit = jnp.any(((v >> b) & 1) == 1)
        out = out + (bit.astype(jnp.int32) << b)
    return out


def _prefix16(m, lanes):
    v = jnp.where(m, 1, 0).astype(jnp.int32)
    for k in (1, 2, 4, 8):
        idx = jnp.maximum(lanes - k, 0)
        sh = v.at[idx].get(mode="promise_in_bounds")
        v = v + jnp.where(lanes >= k, sh, 0)
    return v


def _sc_agg_body(Din, CH, PASSES, x_hbm, src_hbm, dst_hbm, out_hbm,
                 acc, stage, srcbuf, dstbuf, msrc, mdst, sem):
    NCD = Din // 16
    wid = lax.axis_index("s") * 2 + lax.axis_index("c")
    lanes = lax.broadcasted_iota(jnp.int32, (16,), 0)
    zero16 = jnp.zeros((16,), jnp.float32)

    cols = [lanes + 16 * c for c in range(NCD)]

    def flush(off):
        offa = pl.multiple_of(off, 8)
        pltpu.async_copy(x_hbm.at[msrc.at[pl.ds(offa, 64)]], stage, sem).wait()

        def rowbody(r, _):
            r16 = (r // 16) * 16
            dvec = mdst[pl.ds(offa + r16, 16)]
            lsel = jnp.full((16,), r, jnp.int32) & 15
            dsplat = dvec.at[lsel].get(mode="promise_in_bounds")
            for c in range(NCD):
                v = stage[r, pl.ds(16 * c, 16)]
                plsc.addupdate_scatter(acc, [dsplat, cols[c]], v)
            return 0

        lax.fori_loop(0, 64, rowbody, 0)

    for p in range(PASSES):
        chunk = wid * PASSES + p
        lo = chunk * CH

        def zbody(rr, _):
            rrf = jnp.full((16,), rr, jnp.int32)
            for c in range(NCD):
                plsc.store_scatter(acc, [rrf, cols[c]], zero16)
            return 0

        lax.fori_loop(0, CH + 1, zbody, 0)

        def blkbody(blk, cntv):
            eoff = pl.multiple_of(blk * _EB, 8)
            pltpu.async_copy(dst_hbm.at[pl.ds(eoff, _EB)], dstbuf, sem).wait()
            pltpu.async_copy(src_hbm.at[pl.ds(eoff, _EB)], srcbuf, sem).wait()

            def scanbody(i, cntv):
                d = dstbuf[pl.ds(16 * i, 16)]
                m = (d >= lo) & (d < lo + CH)

                @pl.when(jnp.any(m))
                def _():
                    s = srcbuf[pl.ds(16 * i, 16)]
                    pos = cntv + _prefix16(m, lanes) - 1
                    plsc.store_scatter(msrc, [pos], s, mask=m)
                    plsc.store_scatter(mdst, [pos], d - lo, mask=m)

                pc = plsc.all_reduce_population_count(m)
                return cntv + pc

            cntv = lax.fori_loop(0, _EB // 16, scanbody, cntv)
            cnt = _splat_to_scalar(cntv, 12)
            nb = cnt // 64
            lax.fori_loop(0, nb, lambda k, _: (flush(k * 64), 0)[1], 0)
            rb = pl.multiple_of(nb * 64, 8)
            for j in range(4):
                msrc[pl.ds(16 * j, 16)] = msrc[pl.ds(rb + 16 * j, 16)]
                mdst[pl.ds(16 * j, 16)] = mdst[pl.ds(rb + 16 * j, 16)]
            return jnp.full((16,), cnt - nb * 64, jnp.int32)

        cntv = lax.fori_loop(0, _NBLK, blkbody,
                             jnp.zeros((16,), jnp.int32))
        cnt = _splat_to_scalar(cntv, 12)

        for j in range(4):
            pos = cnt + 16 * j + lanes
            plsc.store_scatter(msrc, [pos], jnp.zeros((16,), jnp.int32))
            plsc.store_scatter(mdst, [pos], jnp.full((16,), CH, jnp.int32))
        nb2 = (cnt + 63) // 64
        lax.fori_loop(0, nb2, lambda k, _: (flush(k * 64), 0)[1], 0)

        pltpu.async_copy(acc.at[pl.ds(0, CH)], out_hbm.at[pl.ds(lo, CH)],
                         sem).wait()


@functools.lru_cache(maxsize=None)
def _make_sc_agg(Din, CH, PASSES):
    mesh = plsc.VectorSubcoreMesh(core_axis_name="c", subcore_axis_name="s")
    return pl.kernel(
        functools.partial(_sc_agg_body, Din, CH, PASSES),
        out_type=jax.ShapeDtypeStruct((_NPAD, Din), jnp.float32),
        mesh=mesh,
        compiler_params=pltpu.CompilerParams(use_tc_tiling_on_sc=False, needs_layout_passes=False),
        scratch_types=[
            pltpu.VMEM((CH + 1, Din), jnp.float32),
            pltpu.VMEM((64, Din), jnp.float32),
            pltpu.VMEM((_EB,), jnp.int32),
            pltpu.VMEM((_EB,), jnp.int32),
            pltpu.VMEM((_MB,), jnp.int32),
            pltpu.VMEM((_MB,), jnp.int32),
            pltpu.SemaphoreType.DMA,
        ],
    )


def _agg(x, src, dst):
    d = x.shape[1]
    if d == 256:
        fn = _make_sc_agg(256, 320, 1)
    else:
        fn = _make_sc_agg(512, 160, 2)
    return fn(x, src, dst)[:N]


def _mlpA_body(x_ref, agg_ref, wa_ref, ba_ref, o_ref):
    h = x_ref[...] + agg_ref[...]
    y = jnp.dot(h, wa_ref[...], preferred_element_type=jnp.float32)
    o_ref[...] = jnp.maximum(y + ba_ref[...], 0.0)


def _mlpB_body(h_ref, wb_ref, bb_ref, r_ref, s_ref, q_ref):
    y = jnp.dot(h_ref[...], wb_ref[...], preferred_element_type=jnp.float32)
    r = jnp.maximum(y + bb_ref[...], 0.0)
    r_ref[...] = r

    @pl.when(pl.program_id(0) == 0)
    def _():
        s_ref[...] = jnp.zeros_like(s_ref)
        q_ref[...] = jnp.zeros_like(q_ref)

    s_ref[...] += jnp.sum(r, axis=0, keepdims=True)
    q_ref[...] += jnp.sum(r * r, axis=0, keepdims=True)


def _bn_body(r_ref, s_ref, q_ref, g_ref, be_ref, o_ref):
    mu = s_ref[...] * (1.0 / N)
    var = q_ref[...] * (1.0 / N) - mu * mu
    inv = lax.rsqrt(var + _EPS)
    o_ref[...] = g_ref[...] * ((r_ref[...] - mu) * inv) + be_ref[...]


def _pool_head_body(r_ref, s_ref, q_ref, g_ref, be_ref, ids_ref,
                    w1_ref, b1_ref, w2_ref, b2_ref, o_ref, pool_ref):
    i = pl.program_id(0)
    mu = s_ref[...] * (1.0 / N)
    var = q_ref[...] * (1.0 / N) - mu * mu
    inv = lax.rsqrt(var + _EPS)
    xn = g_ref[...] * ((r_ref[...] - mu) * inv) + be_ref[...]

    @pl.when(i == 0)
    def _():
        pool_ref[...] = jnp.full_like(pool_ref, -jnp.inf)

    idsb = ids_ref[...]
    for g in range(G):
        mask = idsb == g

        @pl.when(jnp.any(mask))
        def _():
            for cc in range(H // 128):
                sel = jnp.where(mask, xn[:, cc * 128:(cc + 1) * 128],
                                -jnp.inf)
                m = jnp.max(sel, axis=0)
                cur = pool_ref[g, pl.ds(cc * 128, 128)]
                pool_ref[g, pl.ds(cc * 128, 128)] = jnp.maximum(cur, m)

    @pl.when(i == NT - 1)
    def _():
        p = pool_ref[...]
        h = jnp.maximum(
            jnp.dot(p, w1_ref[...], preferred_element_type=jnp.float32)
            + b1_ref[...], 0.0)
        o_ref[...] = (
            jnp.dot(h, w2_ref[...], preferred_element_type=jnp.float32)
            + b2_ref[...])


def _row_spec(d):
    return pl.BlockSpec((R, d), lambda i: (i, 0))


def _full_spec(shape):
    nd = len(shape)
    return pl.BlockSpec(shape, lambda i: (0,) * nd)


def _layer(x, agg, Wa, ba, Wb, bb):
    d = x.shape[1]
    h1 = pl.pallas_call(
        _mlpA_body,
        grid=(NT,),
        in_specs=[_row_spec(d), _row_spec(d),
                  _full_spec((d, H)), _full_spec((1, H))],
        out_specs=_row_spec(H),
        out_shape=jax.ShapeDtypeStruct((N, H), jnp.float32),
    )(x, agg, Wa, ba[None])
    r, s, q = pl.pallas_call(
        _mlpB_body,
        grid=(NT,),
        in_specs=[_row_spec(H), _full_spec((H, H)), _full_spec((1, H))],
        out_specs=[_row_spec(H), _full_spec((1, H)), _full_spec((1, H))],
        out_shape=[jax.ShapeDtypeStruct((N, H), jnp.float32),
                   jax.ShapeDtypeStruct((1, H), jnp.float32),
                   jax.ShapeDtypeStruct((1, H), jnp.float32)],
    )(h1, Wb, bb[None])
    return r, s, q


def _bn(r, s, q, g, be):
    return pl.pallas_call(
        _bn_body,
        grid=(NT,),
        in_specs=[_row_spec(H), _full_spec((1, H)), _full_spec((1, H)),
                  _full_spec((1, H)), _full_spec((1, H))],
        out_specs=_row_spec(H),
        out_shape=jax.ShapeDtypeStruct((N, H), jnp.float32),
    )(r, s[None] if s.ndim == 1 else s, q, g[None], be[None])


def _pool_head(r, s, q, g, be, batch, Wf1, bf1, Wf2, bf2):
    C = Wf1.shape[1]
    P = 128
    w1 = jnp.zeros((H, P), jnp.float32).at[:, :C].set(Wf1)
    b1 = jnp.zeros((1, P), jnp.float32).at[0, :C].set(bf1)
    w2 = jnp.zeros((P, P), jnp.float32).at[:C, :C].set(Wf2)
    b2 = jnp.zeros((1, P), jnp.float32).at[0, :C].set(bf2)
    ids = jnp.broadcast_to(batch[:, None], (N, 128))
    out = pl.pallas_call(
        _pool_head_body,
        grid=(NT,),
        in_specs=[_row_spec(H), _full_spec((1, H)), _full_spec((1, H)),
                  _full_spec((1, H)), _full_spec((1, H)),
                  pl.BlockSpec((R, 128), lambda i: (i, 0)),
                  _full_spec((H, P)), _full_spec((1, P)),
                  _full_spec((P, P)), _full_spec((1, P))],
        out_specs=_full_spec((G, P)),
        out_shape=jax.ShapeDtypeStruct((G, P), jnp.float32),
        scratch_shapes=[pltpu.VMEM((G, H), jnp.float32)],
    )(r, s, q, g[None], be[None], ids, w1, b1, w2, b2)
    return out[:, :C]


@jax.jit
def kernel(data_base, edge_index_base, batch_base,
           W1a, b1a, W1b, b1b, g1, be1,
           W2a, b2a, W2b, b2b, g2, be2,
           W3a, b3a, W3b, b3b, g3, be3,
           Wf1, bf1, Wf2, bf2):
    src = edge_index_base[0]
    dst = edge_index_base[1]

    x0 = data_base
    r1, s1, q1 = _layer(x0, _agg(x0, src, dst), W1a, b1a, W1b, b1b)
    x1 = _bn(r1, s1, q1, g1, be1)
    r2, s2, q2 = _layer(x1, _agg(x1, src, dst), W2a, b2a, W2b, b2b)
    x2 = _bn(r2, s2, q2, g2, be2)
    r3, s3, q3 = _layer(x2, _agg(x2, src, dst), W3a, b3a, W3b, b3b)
    return _pool_head(r3, s3, q3, g3, be3, batch_base, Wf1, bf1, Wf2, bf2)


# speedup vs baseline: 1.2382x; 1.2382x over previous
import functools

import jax
import jax.numpy as jnp
from jax import lax
from jax.experimental import pallas as pl
from jax.experimental.pallas import tpu as pltpu
from jax.experimental.pallas import tpu_sc as plsc

N = 10000
E = 160000
H = 512
G = 64
R = 2000
NT = N // R
_EPS = 1e-5


_EB = 2000
_NBLK = E // _EB
_NPAD = 10240
_MB = 2112


def _splat_to_scalar(v, nbits):
    out = jnp.int32(0)
    for b in range(nbits):
        bit = jnp.any(((v >> b) & 1) == 1)
        out = out + (bit.astype(jnp.int32) << b)
    return out


def _prefix16(m, lanes):
    v = jnp.where(m, 1, 0).astype(jnp.int32)
    for k in (1, 2, 4, 8):
        idx = jnp.maximum(lanes - k, 0)
        sh = v.at[idx].get(mode="promise_in_bounds")
        v = v + jnp.where(lanes >= k, sh, 0)
    return v


def _sc_agg_body(Din, CH, PASSES, x_hbm, src_hbm, dst_hbm, out_hbm,
                 acc, stage, srcbuf, dstbuf, msrc, mdst, sem):
    NCD = Din // 16
    wid = lax.axis_index("s") * 2 + lax.axis_index("c")
    lanes = lax.broadcasted_iota(jnp.int32, (16,), 0)
    zero16 = jnp.zeros((16,), jnp.float32)

    cols = [lanes + 16 * c for c in range(NCD)]

    def flush(off):
        offa = pl.multiple_of(off, 8)
        pltpu.async_copy(x_hbm.at[msrc.at[pl.ds(offa, 64)]], stage, sem).wait()

        def rowbody(r, _):
            r16 = (r // 16) * 16
            dvec = mdst[pl.ds(offa + r16, 16)]
            lsel = jnp.full((16,), r, jnp.int32) & 15
            dsplat = dvec.at[lsel].get(mode="promise_in_bounds")
            for c in range(NCD):
                v = stage[r, pl.ds(16 * c, 16)]
                plsc.addupdate_scatter(acc, [dsplat, cols[c]], v)
            return 0

        lax.fori_loop(0, 64, rowbody, 0)

    for p in range(PASSES):
        chunk = wid * PASSES + p
        lo = chunk * CH

        def zbody(rr, _):
            rrf = jnp.full((16,), rr, jnp.int32)
            for c in range(NCD):
                plsc.store_scatter(acc, [rrf, cols[c]], zero16)
            return 0

        lax.fori_loop(0, CH + 1, zbody, 0)

        def blkbody(blk, cntv):
            eoff = pl.multiple_of(blk * _EB, 8)
            pltpu.async_copy(dst_hbm.at[pl.ds(eoff, _EB)], dstbuf, sem).wait()
            pltpu.async_copy(src_hbm.at[pl.ds(eoff, _EB)], srcbuf, sem).wait()

            def scanbody(i, cntv):
                d = dstbuf[pl.ds(16 * i, 16)]
                s = srcbuf[pl.ds(16 * i, 16)]
                m = (d >= lo) & (d < lo + CH)
                pos = cntv + _prefix16(m, lanes) - 1
                plsc.store_scatter(msrc, [pos], s, mask=m)
                plsc.store_scatter(mdst, [pos], d - lo, mask=m)
                pc = plsc.all_reduce_population_count(m)
                return cntv + pc

            cntv = lax.fori_loop(0, _EB // 16, scanbody, cntv)
            cnt = _splat_to_scalar(cntv, 12)
            nb = cnt // 64
            lax.fori_loop(0, nb, lambda k, _: (flush(k * 64), 0)[1], 0)
            rb = pl.multiple_of(nb * 64, 8)
            for j in range(4):
                msrc[pl.ds(16 * j, 16)] = msrc[pl.ds(rb + 16 * j, 16)]
                mdst[pl.ds(16 * j, 16)] = mdst[pl.ds(rb + 16 * j, 16)]
            return jnp.full((16,), cnt - nb * 64, jnp.int32)

        cntv = lax.fori_loop(0, _NBLK, blkbody,
                             jnp.zeros((16,), jnp.int32))
        cnt = _splat_to_scalar(cntv, 12)

        for j in range(4):
            pos = cnt + 16 * j + lanes
            plsc.store_scatter(msrc, [pos], jnp.zeros((16,), jnp.int32))
            plsc.store_scatter(mdst, [pos], jnp.full((16,), CH, jnp.int32))
        nb2 = (cnt + 63) // 64
        lax.fori_loop(0, nb2, lambda k, _: (flush(k * 64), 0)[1], 0)

        pltpu.async_copy(acc.at[pl.ds(0, CH)], out_hbm.at[pl.ds(lo, CH)],
                         sem).wait()


@functools.lru_cache(maxsize=None)
def _make_sc_agg(Din, CH, PASSES):
    mesh = plsc.VectorSubcoreMesh(core_axis_name="c", subcore_axis_name="s")
    return pl.kernel(
        functools.partial(_sc_agg_body, Din, CH, PASSES),
        out_type=jax.ShapeDtypeStruct((_NPAD, Din), jnp.float32),
        mesh=mesh,
        compiler_params=pltpu.CompilerParams(use_tc_tiling_on_sc=False, needs_layout_passes=False),
        scratch_types=[
            pltpu.VMEM((CH + 1, Din), jnp.float32),
            pltpu.VMEM((64, Din), jnp.float32),
            pltpu.VMEM((_EB,), jnp.int32),
            pltpu.VMEM((_EB,), jnp.int32),
            pltpu.VMEM((_MB,), jnp.int32),
            pltpu.VMEM((_MB,), jnp.int32),
            pltpu.SemaphoreType.DMA,
        ],
    )


def _agg(x, src, dst):
    d = x.shape[1]
    if d == 256:
        fn = _make_sc_agg(256, 320, 1)
    else:
        fn = _make_sc_agg(512, 160, 2)
    return fn(x, src, dst)[:N]


def _mlpA_body(x_ref, agg_ref, wa_ref, ba_ref, o_ref):
    h = x_ref[...] + agg_ref[...]
    y = jnp.dot(h, wa_ref[...], preferred_element_type=jnp.float32)
    o_ref[...] = jnp.maximum(y + ba_ref[...], 0.0)


def _mlpB_body(h_ref, wb_ref, bb_ref, r_ref, s_ref, q_ref):
    y = jnp.dot(h_ref[...], wb_ref[...], preferred_element_type=jnp.float32)
    r = jnp.maximum(y + bb_ref[...], 0.0)
    r_ref[...] = r

    @pl.when(pl.program_id(0) == 0)
    def _():
        s_ref[...] = jnp.zeros_like(s_ref)
        q_ref[...] = jnp.zeros_like(q_ref)

    s_ref[...] += jnp.sum(r, axis=0, keepdims=True)
    q_ref[...] += jnp.sum(r * r, axis=0, keepdims=True)


def _bn_body(r_ref, s_ref, q_ref, g_ref, be_ref, o_ref):
    mu = s_ref[...] * (1.0 / N)
    var = q_ref[...] * (1.0 / N) - mu * mu
    inv = lax.rsqrt(var + _EPS)
    o_ref[...] = g_ref[...] * ((r_ref[...] - mu) * inv) + be_ref[...]


def _pool_head_body(r_ref, s_ref, q_ref, g_ref, be_ref, ids_ref,
                    w1_ref, b1_ref, w2_ref, b2_ref, o_ref, pool_ref):
    i = pl.program_id(0)
    mu = s_ref[...] * (1.0 / N)
    var = q_ref[...] * (1.0 / N) - mu * mu
    inv = lax.rsqrt(var + _EPS)
    xn = g_ref[...] * ((r_ref[...] - mu) * inv) + be_ref[...]

    @pl.when(i == 0)
    def _():
        pool_ref[...] = jnp.full_like(pool_ref, -jnp.inf)

    idsb = ids_ref[...]
    for g in range(G):
        mask = idsb == g

        @pl.when(jnp.any(mask))
        def _():
            for cc in range(H // 128):
                sel = jnp.where(mask, xn[:, cc * 128:(cc + 1) * 128],
                                -jnp.inf)
                m = jnp.max(sel, axis=0)
                cur = pool_ref[g, pl.ds(cc * 128, 128)]
                pool_ref[g, pl.ds(cc * 128, 128)] = jnp.maximum(cur, m)

    @pl.when(i == NT - 1)
    def _():
        p = pool_ref[...]
        h = jnp.maximum(
            jnp.dot(p, w1_ref[...], preferred_element_type=jnp.float32)
            + b1_ref[...], 0.0)
        o_ref[...] = (
            jnp.dot(h, w2_ref[...], preferred_element_type=jnp.float32)
            + b2_ref[...])


def _row_spec(d):
    return pl.BlockSpec((R, d), lambda i: (i, 0))


def _full_spec(shape):
    nd = len(shape)
    return pl.BlockSpec(shape, lambda i: (0,) * nd)


def _layer(x, agg, Wa, ba, Wb, bb):
    d = x.shape[1]
    h1 = pl.pallas_call(
        _mlpA_body,
        grid=(NT,),
        in_specs=[_row_spec(d), _row_spec(d),
                  _full_spec((d, H)), _full_spec((1, H))],
        out_specs=_row_spec(H),
        out_shape=jax.ShapeDtypeStruct((N, H), jnp.float32),
    )(x, agg, Wa, ba[None])
    r, s, q = pl.pallas_call(
        _mlpB_body,
        grid=(NT,),
        in_specs=[_row_spec(H), _full_spec((H, H)), _full_spec((1, H))],
        out_specs=[_row_spec(H), _full_spec((1, H)), _full_spec((1, H))],
        out_shape=[jax.ShapeDtypeStruct((N, H), jnp.float32),
                   jax.ShapeDtypeStruct((1, H), jnp.float32),
                   jax.ShapeDtypeStruct((1, H), jnp.float32)],
    )(h1, Wb, bb[None])
    return r, s, q


def _bn(r, s, q, g, be):
    return pl.pallas_call(
        _bn_body,
        grid=(NT,),
        in_specs=[_row_spec(H), _full_spec((1, H)), _full_spec((1, H)),
                  _full_spec((1, H)), _full_spec((1, H))],
        out_specs=_row_spec(H),
        out_shape=jax.ShapeDtypeStruct((N, H), jnp.float32),
    )(r, s[None] if s.ndim == 1 else s, q, g[None], be[None])


def _pool_head(r, s, q, g, be, batch, Wf1, bf1, Wf2, bf2):
    C = Wf1.shape[1]
    P = 128
    w1 = jnp.zeros((H, P), jnp.float32).at[:, :C].set(Wf1)
    b1 = jnp.zeros((1, P), jnp.float32).at[0, :C].set(bf1)
    w2 = jnp.zeros((P, P), jnp.float32).at[:C, :C].set(Wf2)
    b2 = jnp.zeros((1, P), jnp.float32).at[0, :C].set(bf2)
    ids = jnp.broadcast_to(batch[:, None], (N, 128))
    out = pl.pallas_call(
        _pool_head_body,
        grid=(NT,),
        in_specs=[_row_spec(H), _full_spec((1, H)), _full_spec((1, H)),
                  _full_spec((1, H)), _full_spec((1, H)),
                  pl.BlockSpec((R, 128), lambda i: (i, 0)),
                  _full_spec((H, P)), _full_spec((1, P)),
                  _full_spec((P, P)), _full_spec((1, P))],
        out_specs=_full_spec((G, P)),
        out_shape=jax.ShapeDtypeStruct((G, P), jnp.float32),
        scratch_shapes=[pltpu.VMEM((G, H), jnp.float32)],
    )(r, s, q, g[None], be[None], ids, w1, b1, w2, b2)
    return out[:, :C]


@jax.jit
def kernel(data_base, edge_index_base, batch_base,
           W1a, b1a, W1b, b1b, g1, be1,
           W2a, b2a, W2b, b2b, g2, be2,
           W3a, b3a, W3b, b3b, g3, be3,
           Wf1, bf1, Wf2, bf2):
    src = edge_index_base[0]
    dst = edge_index_base[1]

    x0 = data_base
    r1, s1, q1 = _layer(x0, _agg(x0, src, dst), W1a, b1a, W1b, b1b)
    x1 = _bn(r1, s1, q1, g1, be1)
    r2, s2, q2 = _layer(x1, _agg(x1, src, dst), W2a, b2a, W2b, b2b)
    x2 = _bn(r2, s2, q2, g2, be2)
    r3, s3, q3 = _layer(x2, _agg(x2, src, dst), W3a, b3a, W3b, b3b)
    return _pool_head(r3, s3, q3, g3, be3, batch_base, Wf1, bf1, Wf2, bf2)


# speedup vs baseline: 1.4091x; 1.1380x over previous
import functools

import jax
import jax.numpy as jnp
from jax import lax
from jax.experimental import pallas as pl
from jax.experimental.pallas import tpu as pltpu
from jax.experimental.pallas import tpu_sc as plsc

N = 10000
E = 160000
H = 512
G = 64
R = 2000
NT = N // R
_EPS = 1e-5


_EB = 2000
_NBLK = E // _EB
_NPAD = 10240
_MB = 2112


def _splat_to_scalar(v, nbits):
    out = jnp.int32(0)
    for b in range(nbits):
        bit = jnp.any(((v >> b) & 1) == 1)
        out = out + (bit.astype(jnp.int32) << b)
    return out


def _prefix16(m, lanes):
    v = jnp.where(m, 1, 0).astype(jnp.int32)
    for k in (1, 2, 4, 8):
        idx = jnp.maximum(lanes - k, 0)
        sh = v.at[idx].get(mode="promise_in_bounds")
        v = v + jnp.where(lanes >= k, sh, 0)
    return v


def _sc_agg_body(Din, CH, PASSES, x_hbm, src_hbm, dst_hbm, out_hbm,
                 acc, stage, srcbuf, dstbuf, msrc, mdst, sem_e, sem_g):
    NCD = Din // 16
    wid = lax.axis_index("s") * 2 + lax.axis_index("c")
    lanes = lax.broadcasted_iota(jnp.int32, (16,), 0)
    zero16 = jnp.zeros((16,), jnp.float32)

    cols = [lanes + 16 * c for c in range(NCD)]

    def fire_edge(blk, par):
        eoff = pl.multiple_of(blk * _EB, 8)
        pltpu.async_copy(dst_hbm.at[pl.ds(eoff, _EB)], dstbuf.at[par], sem_e)
        pltpu.async_copy(src_hbm.at[pl.ds(eoff, _EB)], srcbuf.at[par], sem_e)

    def wait_edge(par):
        pltpu.make_async_copy(dst_hbm.at[pl.ds(0, _EB)], dstbuf.at[par],
                              sem_e).wait()
        pltpu.make_async_copy(src_hbm.at[pl.ds(0, _EB)], srcbuf.at[par],
                              sem_e).wait()

    def flush(off):
        offa = pl.multiple_of(off, 8)
        pltpu.async_copy(x_hbm.at[msrc.at[pl.ds(offa, 64)]], stage,
                         sem_g).wait()

        def rowbody(r, _):
            r16 = (r // 16) * 16
            dvec = mdst[pl.ds(offa + r16, 16)]
            lsel = jnp.full((16,), r, jnp.int32) & 15
            dsplat = dvec.at[lsel].get(mode="promise_in_bounds")
            for c in range(NCD):
                v = stage[r, pl.ds(16 * c, 16)]
                plsc.addupdate_scatter(acc, [dsplat, cols[c]], v)
            return 0

        lax.fori_loop(0, 64, rowbody, 0)

    for p in range(PASSES):
        chunk = wid * PASSES + p
        lo = chunk * CH

        def zbody(rr, _):
            rrf = jnp.full((16,), rr, jnp.int32)
            for c in range(NCD):
                plsc.store_scatter(acc, [rrf, cols[c]], zero16)
            return 0

        lax.fori_loop(0, CH + 1, zbody, 0)

        def scanflush(par, cntv):
            def scanbody(i, cntv):
                d = dstbuf[par, pl.ds(16 * i, 16)]
                s = srcbuf[par, pl.ds(16 * i, 16)]
                m = (d >= lo) & (d < lo + CH)
                pos = cntv + _prefix16(m, lanes) - 1
                plsc.store_scatter(msrc, [pos], s, mask=m)
                plsc.store_scatter(mdst, [pos], d - lo, mask=m)
                pc = plsc.all_reduce_population_count(m)
                return cntv + pc

            cntv = lax.fori_loop(0, _EB // 16, scanbody, cntv)
            cnt = _splat_to_scalar(cntv, 12)
            nb = cnt // 64
            lax.fori_loop(0, nb, lambda k, _: (flush(k * 64), 0)[1], 0)
            rb = pl.multiple_of(nb * 64, 8)
            for j in range(4):
                msrc[pl.ds(16 * j, 16)] = msrc[pl.ds(rb + 16 * j, 16)]
                mdst[pl.ds(16 * j, 16)] = mdst[pl.ds(rb + 16 * j, 16)]
            return jnp.full((16,), cnt - nb * 64, jnp.int32)

        fire_edge(0, 0)

        def pairbody(j, cntv):
            fire_edge(2 * j + 1, 1)
            wait_edge(0)
            cntv = scanflush(0, cntv)

            @pl.when(j < _NBLK // 2 - 1)
            def _():
                fire_edge(2 * j + 2, 0)

            wait_edge(1)
            cntv = scanflush(1, cntv)
            return cntv

        cntv = lax.fori_loop(0, _NBLK // 2, pairbody,
                             jnp.zeros((16,), jnp.int32))
        cnt = _splat_to_scalar(cntv, 12)

        for j in range(4):
            pos = cnt + 16 * j + lanes
            plsc.store_scatter(msrc, [pos], jnp.zeros((16,), jnp.int32))
            plsc.store_scatter(mdst, [pos], jnp.full((16,), CH, jnp.int32))
        nb2 = (cnt + 63) // 64
        lax.fori_loop(0, nb2, lambda k, _: (flush(k * 64), 0)[1], 0)

        pltpu.async_copy(acc.at[pl.ds(0, CH)], out_hbm.at[pl.ds(lo, CH)],
                         sem_g).wait()


@functools.lru_cache(maxsize=None)
def _make_sc_agg(Din, CH, PASSES):
    mesh = plsc.VectorSubcoreMesh(core_axis_name="c", subcore_axis_name="s")
    return pl.kernel(
        functools.partial(_sc_agg_body, Din, CH, PASSES),
        out_type=jax.ShapeDtypeStruct((_NPAD, Din), jnp.float32),
        mesh=mesh,
        compiler_params=pltpu.CompilerParams(use_tc_tiling_on_sc=False, needs_layout_passes=False),
        scratch_types=[
            pltpu.VMEM((CH + 1, Din), jnp.float32),
            pltpu.VMEM((64, Din), jnp.float32),
            pltpu.VMEM((2, _EB), jnp.int32),
            pltpu.VMEM((2, _EB), jnp.int32),
            pltpu.VMEM((_MB,), jnp.int32),
            pltpu.VMEM((_MB,), jnp.int32),
            pltpu.SemaphoreType.DMA,
            pltpu.SemaphoreType.DMA,
        ],
    )


def _agg(x, src, dst):
    d = x.shape[1]
    if d == 256:
        fn = _make_sc_agg(256, 320, 1)
    else:
        fn = _make_sc_agg(512, 160, 2)
    return fn(x, src, dst)[:N]


def _mlpA_body(x_ref, agg_ref, wa_ref, ba_ref, o_ref):
    h = x_ref[...] + agg_ref[...]
    y = jnp.dot(h, wa_ref[...], preferred_element_type=jnp.float32)
    o_ref[...] = jnp.maximum(y + ba_ref[...], 0.0)


def _mlpB_body(h_ref, wb_ref, bb_ref, r_ref, s_ref, q_ref):
    y = jnp.dot(h_ref[...], wb_ref[...], preferred_element_type=jnp.float32)
    r = jnp.maximum(y + bb_ref[...], 0.0)
    r_ref[...] = r

    @pl.when(pl.program_id(0) == 0)
    def _():
        s_ref[...] = jnp.zeros_like(s_ref)
        q_ref[...] = jnp.zeros_like(q_ref)

    s_ref[...] += jnp.sum(r, axis=0, keepdims=True)
    q_ref[...] += jnp.sum(r * r, axis=0, keepdims=True)


def _bn_body(r_ref, s_ref, q_ref, g_ref, be_ref, o_ref):
    mu = s_ref[...] * (1.0 / N)
    var = q_ref[...] * (1.0 / N) - mu * mu
    inv = lax.rsqrt(var + _EPS)
    o_ref[...] = g_ref[...] * ((r_ref[...] - mu) * inv) + be_ref[...]


def _pool_head_body(r_ref, s_ref, q_ref, g_ref, be_ref, ids_ref,
                    w1_ref, b1_ref, w2_ref, b2_ref, o_ref, pool_ref):
    i = pl.program_id(0)
    mu = s_ref[...] * (1.0 / N)
    var = q_ref[...] * (1.0 / N) - mu * mu
    inv = lax.rsqrt(var + _EPS)
    xn = g_ref[...] * ((r_ref[...] - mu) * inv) + be_ref[...]

    @pl.when(i == 0)
    def _():
        pool_ref[...] = jnp.full_like(pool_ref, -jnp.inf)

    idsb = ids_ref[...]
    for g in range(G):
        mask = idsb == g

        @pl.when(jnp.any(mask))
        def _():
            for cc in range(H // 128):
                sel = jnp.where(mask, xn[:, cc * 128:(cc + 1) * 128],
                                -jnp.inf)
                m = jnp.max(sel, axis=0)
                cur = pool_ref[g, pl.ds(cc * 128, 128)]
                pool_ref[g, pl.ds(cc * 128, 128)] = jnp.maximum(cur, m)

    @pl.when(i == NT - 1)
    def _():
        p = pool_ref[...]
        h = jnp.maximum(
            jnp.dot(p, w1_ref[...], preferred_element_type=jnp.float32)
            + b1_ref[...], 0.0)
        o_ref[...] = (
            jnp.dot(h, w2_ref[...], preferred_element_type=jnp.float32)
            + b2_ref[...])


def _row_spec(d):
    return pl.BlockSpec((R, d), lambda i: (i, 0))


def _full_spec(shape):
    nd = len(shape)
    return pl.BlockSpec(shape, lambda i: (0,) * nd)


def _layer(x, agg, Wa, ba, Wb, bb):
    d = x.shape[1]
    h1 = pl.pallas_call(
        _mlpA_body,
        grid=(NT,),
        in_specs=[_row_spec(d), _row_spec(d),
                  _full_spec((d, H)), _full_spec((1, H))],
        out_specs=_row_spec(H),
        out_shape=jax.ShapeDtypeStruct((N, H), jnp.float32),
    )(x, agg, Wa, ba[None])
    r, s, q = pl.pallas_call(
        _mlpB_body,
        grid=(NT,),
        in_specs=[_row_spec(H), _full_spec((H, H)), _full_spec((1, H))],
        out_specs=[_row_spec(H), _full_spec((1, H)), _full_spec((1, H))],
        out_shape=[jax.ShapeDtypeStruct((N, H), jnp.float32),
                   jax.ShapeDtypeStruct((1, H), jnp.float32),
                   jax.ShapeDtypeStruct((1, H), jnp.float32)],
    )(h1, Wb, bb[None])
    return r, s, q


def _bn(r, s, q, g, be):
    return pl.pallas_call(
        _bn_body,
        grid=(NT,),
        in_specs=[_row_spec(H), _full_spec((1, H)), _full_spec((1, H)),
                  _full_spec((1, H)), _full_spec((1, H))],
        out_specs=_row_spec(H),
        out_shape=jax.ShapeDtypeStruct((N, H), jnp.float32),
    )(r, s[None] if s.ndim == 1 else s, q, g[None], be[None])


def _pool_head(r, s, q, g, be, batch, Wf1, bf1, Wf2, bf2):
    C = Wf1.shape[1]
    P = 128
    w1 = jnp.zeros((H, P), jnp.float32).at[:, :C].set(Wf1)
    b1 = jnp.zeros((1, P), jnp.float32).at[0, :C].set(bf1)
    w2 = jnp.zeros((P, P), jnp.float32).at[:C, :C].set(Wf2)
    b2 = jnp.zeros((1, P), jnp.float32).at[0, :C].set(bf2)
    ids = jnp.broadcast_to(batch[:, None], (N, 128))
    out = pl.pallas_call(
        _pool_head_body,
        grid=(NT,),
        in_specs=[_row_spec(H), _full_spec((1, H)), _full_spec((1, H)),
                  _full_spec((1, H)), _full_spec((1, H)),
                  pl.BlockSpec((R, 128), lambda i: (i, 0)),
                  _full_spec((H, P)), _full_spec((1, P)),
                  _full_spec((P, P)), _full_spec((1, P))],
        out_specs=_full_spec((G, P)),
        out_shape=jax.ShapeDtypeStruct((G, P), jnp.float32),
        scratch_shapes=[pltpu.VMEM((G, H), jnp.float32)],
    )(r, s, q, g[None], be[None], ids, w1, b1, w2, b2)
    return out[:, :C]


@jax.jit
def kernel(data_base, edge_index_base, batch_base,
           W1a, b1a, W1b, b1b, g1, be1,
           W2a, b2a, W2b, b2b, g2, be2,
           W3a, b3a, W3b, b3b, g3, be3,
           Wf1, bf1, Wf2, bf2):
    src = edge_index_base[0]
    dst = edge_index_base[1]

    x0 = data_base
    r1, s1, q1 = _layer(x0, _agg(x0, src, dst), W1a, b1a, W1b, b1b)
    x1 = _bn(r1, s1, q1, g1, be1)
    r2, s2, q2 = _layer(x1, _agg(x1, src, dst), W2a, b2a, W2b, b2b)
    x2 = _bn(r2, s2, q2, g2, be2)
    r3, s3, q3 = _layer(x2, _agg(x2, src, dst), W3a, b3a, W3b, b3b)
    return _pool_head(r3, s3, q3, g3, be3, batch_base, Wf1, bf1, Wf2, bf2)


# speedup vs baseline: 1.8312x; 1.2996x over previous
import functools

import jax
import jax.numpy as jnp
from jax import lax
from jax.experimental import pallas as pl
from jax.experimental.pallas import tpu as pltpu
from jax.experimental.pallas import tpu_sc as plsc

N = 10000
E = 160000
H = 512
G = 64
R = 2000
NT = N // R
_EPS = 1e-5


_EB = 2000
_NBLK = E // _EB
_NPAD = 10240
_MB = 2112


def _splat_to_scalar(v, nbits):
    out = jnp.int32(0)
    for b in range(nbits):
        bit = jnp.any(((v >> b) & 1) == 1)
        out = out + (bit.astype(jnp.int32) << b)
    return out


def _prefix16(m, lanes):
    v = jnp.where(m, 1, 0).astype(jnp.int32)
    for k in (1, 2, 4, 8):
        idx = jnp.maximum(lanes - k, 0)
        sh = v.at[idx].get(mode="promise_in_bounds")
        v = v + jnp.where(lanes >= k, sh, 0)
    return v


def _sc_agg_body(Din, CH, PASSES, x_hbm, src_hbm, dst_hbm, out_hbm,
                 acc, stg0, stg1, srcbuf, dstbuf, msrc, mdst, sem_e, sem_g):
    NCD = Din // 16
    wid = lax.axis_index("s") * 2 + lax.axis_index("c")
    lanes = lax.broadcasted_iota(jnp.int32, (16,), 0)
    zero16 = jnp.zeros((16,), jnp.float32)

    cols = [lanes + 16 * c for c in range(NCD)]

    def fire_edge(blk, par):
        eoff = pl.multiple_of(blk * _EB, 8)
        pltpu.async_copy(dst_hbm.at[pl.ds(eoff, _EB)], dstbuf.at[par], sem_e)
        pltpu.async_copy(src_hbm.at[pl.ds(eoff, _EB)], srcbuf.at[par], sem_e)

    def wait_edge(par):
        pltpu.make_async_copy(dst_hbm.at[pl.ds(0, _EB)], dstbuf.at[par],
                              sem_e).wait()
        pltpu.make_async_copy(src_hbm.at[pl.ds(0, _EB)], srcbuf.at[par],
                              sem_e).wait()

    def fire_g(kb, stg):
        offb = pl.multiple_of(kb * 32, 8)
        pltpu.async_copy(x_hbm.at[msrc.at[pl.ds(offb, 32)]], stg, sem_g)

    def wait_g(stg):
        pltpu.make_async_copy(x_hbm.at[pl.ds(0, 32)], stg, sem_g).wait()

    def accum(stg, kb):
        base = kb * 32

        def rowbody(r, _):
            r16 = (r // 16) * 16
            dvec = mdst[pl.ds(base + r16, 16)]
            lsel = jnp.full((16,), r, jnp.int32) & 15
            dsplat = dvec.at[lsel].get(mode="promise_in_bounds")
            for c in range(NCD):
                v = stg[r, pl.ds(16 * c, 16)]
                plsc.addupdate_scatter(acc, [dsplat, cols[c]], v)
            return 0

        lax.fori_loop(0, 32, rowbody, 0)

    def flush_range(nb64):
        @pl.when(nb64 > 0)
        def _():
            fire_g(0, stg0)

            def fpair(jp, _):
                fire_g(2 * jp + 1, stg1)
                wait_g(stg0)
                accum(stg0, 2 * jp)

                @pl.when(jp < nb64 - 1)
                def _():
                    fire_g(2 * jp + 2, stg0)

                wait_g(stg1)
                accum(stg1, 2 * jp + 1)
                return 0

            lax.fori_loop(0, nb64, fpair, 0)

    for p in range(PASSES):
        chunk = wid * PASSES + p
        lo = chunk * CH

        def zbody(rr, _):
            rrf = jnp.full((16,), rr, jnp.int32)
            for c in range(NCD):
                plsc.store_scatter(acc, [rrf, cols[c]], zero16)
            return 0

        lax.fori_loop(0, CH + 1, zbody, 0)

        def scanflush(par, cntv):
            def scanbody(i, cntv):
                d = dstbuf[par, pl.ds(16 * i, 16)]
                s = srcbuf[par, pl.ds(16 * i, 16)]
                m = (d >= lo) & (d < lo + CH)
                pos = cntv + _prefix16(m, lanes) - 1
                plsc.store_scatter(msrc, [pos], s, mask=m)
                plsc.store_scatter(mdst, [pos], d - lo, mask=m)
                pc = plsc.all_reduce_population_count(m)
                return cntv + pc

            cntv = lax.fori_loop(0, _EB // 16, scanbody, cntv)
            cnt = _splat_to_scalar(cntv, 12)
            nb = cnt // 64
            flush_range(nb)
            rb = pl.multiple_of(nb * 64, 8)
            for j in range(4):
                msrc[pl.ds(16 * j, 16)] = msrc[pl.ds(rb + 16 * j, 16)]
                mdst[pl.ds(16 * j, 16)] = mdst[pl.ds(rb + 16 * j, 16)]
            return jnp.full((16,), cnt - nb * 64, jnp.int32)

        fire_edge(0, 0)

        def pairbody(j, cntv):
            fire_edge(2 * j + 1, 1)
            wait_edge(0)
            cntv = scanflush(0, cntv)

            @pl.when(j < _NBLK // 2 - 1)
            def _():
                fire_edge(2 * j + 2, 0)

            wait_edge(1)
            cntv = scanflush(1, cntv)
            return cntv

        cntv = lax.fori_loop(0, _NBLK // 2, pairbody,
                             jnp.zeros((16,), jnp.int32))
        cnt = _splat_to_scalar(cntv, 12)

        for j in range(4):
            pos = cnt + 16 * j + lanes
            plsc.store_scatter(msrc, [pos], jnp.zeros((16,), jnp.int32))
            plsc.store_scatter(mdst, [pos], jnp.full((16,), CH, jnp.int32))
        nb2 = (cnt + 63) // 64
        flush_range(nb2)

        pltpu.async_copy(acc.at[pl.ds(0, CH)], out_hbm.at[pl.ds(lo, CH)],
                         sem_g).wait()


@functools.lru_cache(maxsize=None)
def _make_sc_agg(Din, CH, PASSES):
    mesh = plsc.VectorSubcoreMesh(core_axis_name="c", subcore_axis_name="s")
    return pl.kernel(
        functools.partial(_sc_agg_body, Din, CH, PASSES),
        out_type=jax.ShapeDtypeStruct((_NPAD, Din), jnp.float32),
        mesh=mesh,
        compiler_params=pltpu.CompilerParams(use_tc_tiling_on_sc=False, needs_layout_passes=False),
        scratch_types=[
            pltpu.VMEM((CH + 1, Din), jnp.float32),
            pltpu.VMEM((32, Din), jnp.float32),
            pltpu.VMEM((32, Din), jnp.float32),
            pltpu.VMEM((2, _EB), jnp.int32),
            pltpu.VMEM((2, _EB), jnp.int32),
            pltpu.VMEM((_MB,), jnp.int32),
            pltpu.VMEM((_MB,), jnp.int32),
            pltpu.SemaphoreType.DMA,
            pltpu.SemaphoreType.DMA,
        ],
    )


def _agg(x, src, dst):
    d = x.shape[1]
    if d == 256:
        fn = _make_sc_agg(256, 320, 1)
    else:
        fn = _make_sc_agg(512, 160, 2)
    return fn(x, src, dst)[:N]


def _mlpA_body(x_ref, agg_ref, wa_ref, ba_ref, o_ref):
    h = x_ref[...] + agg_ref[...]
    y = jnp.dot(h, wa_ref[...], preferred_element_type=jnp.float32)
    o_ref[...] = jnp.maximum(y + ba_ref[...], 0.0)


def _mlpB_body(h_ref, wb_ref, bb_ref, r_ref, s_ref, q_ref):
    y = jnp.dot(h_ref[...], wb_ref[...], preferred_element_type=jnp.float32)
    r = jnp.maximum(y + bb_ref[...], 0.0)
    r_ref[...] = r

    @pl.when(pl.program_id(0) == 0)
    def _():
        s_ref[...] = jnp.zeros_like(s_ref)
        q_ref[...] = jnp.zeros_like(q_ref)

    s_ref[...] += jnp.sum(r, axis=0, keepdims=True)
    q_ref[...] += jnp.sum(r * r, axis=0, keepdims=True)


def _bn_body(r_ref, s_ref, q_ref, g_ref, be_ref, o_ref):
    mu = s_ref[...] * (1.0 / N)
    var = q_ref[...] * (1.0 / N) - mu * mu
    inv = lax.rsqrt(var + _EPS)
    o_ref[...] = g_ref[...] * ((r_ref[...] - mu) * inv) + be_ref[...]


def _pool_head_body(r_ref, s_ref, q_ref, g_ref, be_ref, ids_ref,
                    w1_ref, b1_ref, w2_ref, b2_ref, o_ref, pool_ref):
    i = pl.program_id(0)
    mu = s_ref[...] * (1.0 / N)
    var = q_ref[...] * (1.0 / N) - mu * mu
    inv = lax.rsqrt(var + _EPS)
    xn = g_ref[...] * ((r_ref[...] - mu) * inv) + be_ref[...]

    @pl.when(i == 0)
    def _():
        pool_ref[...] = jnp.full_like(pool_ref, -jnp.inf)

    idsb = ids_ref[...]
    for g in range(G):
        mask = idsb == g

        @pl.when(jnp.any(mask))
        def _():
            for cc in range(H // 128):
                sel = jnp.where(mask, xn[:, cc * 128:(cc + 1) * 128],
                                -jnp.inf)
                m = jnp.max(sel, axis=0)
                cur = pool_ref[g, pl.ds(cc * 128, 128)]
                pool_ref[g, pl.ds(cc * 128, 128)] = jnp.maximum(cur, m)

    @pl.when(i == NT - 1)
    def _():
        p = pool_ref[...]
        h = jnp.maximum(
            jnp.dot(p, w1_ref[...], preferred_element_type=jnp.float32)
            + b1_ref[...], 0.0)
        o_ref[...] = (
            jnp.dot(h, w2_ref[...], preferred_element_type=jnp.float32)
            + b2_ref[...])


def _row_spec(d):
    return pl.BlockSpec((R, d), lambda i: (i, 0))


def _full_spec(shape):
    nd = len(shape)
    return pl.BlockSpec(shape, lambda i: (0,) * nd)


def _layer(x, agg, Wa, ba, Wb, bb):
    d = x.shape[1]
    h1 = pl.pallas_call(
        _mlpA_body,
        grid=(NT,),
        in_specs=[_row_spec(d), _row_spec(d),
                  _full_spec((d, H)), _full_spec((1, H))],
        out_specs=_row_spec(H),
        out_shape=jax.ShapeDtypeStruct((N, H), jnp.float32),
    )(x, agg, Wa, ba[None])
    r, s, q = pl.pallas_call(
        _mlpB_body,
        grid=(NT,),
        in_specs=[_row_spec(H), _full_spec((H, H)), _full_spec((1, H))],
        out_specs=[_row_spec(H), _full_spec((1, H)), _full_spec((1, H))],
        out_shape=[jax.ShapeDtypeStruct((N, H), jnp.float32),
                   jax.ShapeDtypeStruct((1, H), jnp.float32),
                   jax.ShapeDtypeStruct((1, H), jnp.float32)],
    )(h1, Wb, bb[None])
    return r, s, q


def _bn(r, s, q, g, be):
    return pl.pallas_call(
        _bn_body,
        grid=(NT,),
        in_specs=[_row_spec(H), _full_spec((1, H)), _full_spec((1, H)),
                  _full_spec((1, H)), _full_spec((1, H))],
        out_specs=_row_spec(H),
        out_shape=jax.ShapeDtypeStruct((N, H), jnp.float32),
    )(r, s[None] if s.ndim == 1 else s, q, g[None], be[None])


def _pool_head(r, s, q, g, be, batch, Wf1, bf1, Wf2, bf2):
    C = Wf1.shape[1]
    P = 128
    w1 = jnp.zeros((H, P), jnp.float32).at[:, :C].set(Wf1)
    b1 = jnp.zeros((1, P), jnp.float32).at[0, :C].set(bf1)
    w2 = jnp.zeros((P, P), jnp.float32).at[:C, :C].set(Wf2)
    b2 = jnp.zeros((1, P), jnp.float32).at[0, :C].set(bf2)
    ids = jnp.broadcast_to(batch[:, None], (N, 128))
    out = pl.pallas_call(
        _pool_head_body,
        grid=(NT,),
        in_specs=[_row_spec(H), _full_spec((1, H)), _full_spec((1, H)),
                  _full_spec((1, H)), _full_spec((1, H)),
                  pl.BlockSpec((R, 128), lambda i: (i, 0)),
                  _full_spec((H, P)), _full_spec((1, P)),
                  _full_spec((P, P)), _full_spec((1, P))],
        out_specs=_full_spec((G, P)),
        out_shape=jax.ShapeDtypeStruct((G, P), jnp.float32),
        scratch_shapes=[pltpu.VMEM((G, H), jnp.float32)],
    )(r, s, q, g[None], be[None], ids, w1, b1, w2, b2)
    return out[:, :C]


@jax.jit
def kernel(data_base, edge_index_base, batch_base,
           W1a, b1a, W1b, b1b, g1, be1,
           W2a, b2a, W2b, b2b, g2, be2,
           W3a, b3a, W3b, b3b, g3, be3,
           Wf1, bf1, Wf2, bf2):
    src = edge_index_base[0]
    dst = edge_index_base[1]

    x0 = data_base
    r1, s1, q1 = _layer(x0, _agg(x0, src, dst), W1a, b1a, W1b, b1b)
    x1 = _bn(r1, s1, q1, g1, be1)
    r2, s2, q2 = _layer(x1, _agg(x1, src, dst), W2a, b2a, W2b, b2b)
    x2 = _bn(r2, s2, q2, g2, be2)
    r3, s3, q3 = _layer(x2, _agg(x2, src, dst), W3a, b3a, W3b, b3b)
    return _pool_head(r3, s3, q3, g3, be3, batch_base, Wf1, bf1, Wf2, bf2)


# speedup vs baseline: 1.8317x; 1.0003x over previous
import functools

import jax
import jax.numpy as jnp
from jax import lax
from jax.experimental import pallas as pl
from jax.experimental.pallas import tpu as pltpu
from jax.experimental.pallas import tpu_sc as plsc

N = 10000
E = 160000
H = 512
G = 64
R = 2000
NT = N // R
_EPS = 1e-5


_EB = 2000
_NBLK = E // _EB
_NPAD = 10240
_MB = 2112


def _splat_to_scalar(v, nbits):
    out = jnp.int32(0)
    for b in range(nbits):
        bit = jnp.any(((v >> b) & 1) == 1)
        out = out + (bit.astype(jnp.int32) << b)
    return out


def _prefix16(m, lanes):
    v = jnp.where(m, 1, 0).astype(jnp.int32)
    for k in (1, 2, 4, 8):
        idx = jnp.maximum(lanes - k, 0)
        sh = v.at[idx].get(mode="promise_in_bounds")
        v = v + jnp.where(lanes >= k, sh, 0)
    return v


_CH = 160
_STRIDE = E + 2048
_MBB = 4160


def _sc_build_body(x_hbm, src_hbm, dst_hbm, out_hbm, lsrc_hbm, ldst_hbm,
                   cnts_hbm, acc, stg0, stg1, srcbuf, dstbuf, msrc, mdst,
                   cbuf, sem_e, sem_g, sem_s):
    Din = 256
    NCD = Din // 16
    wid = lax.axis_index("s") * 2 + lax.axis_index("c")
    lanes = lax.broadcasted_iota(jnp.int32, (16,), 0)
    zero16 = jnp.zeros((16,), jnp.float32)
    cols = [lanes + 16 * c for c in range(NCD)]

    def fire_edge(blk, par):
        eoff = pl.multiple_of(blk * _EB, 8)
        pltpu.async_copy(dst_hbm.at[pl.ds(eoff, _EB)], dstbuf.at[par], sem_e)
        pltpu.async_copy(src_hbm.at[pl.ds(eoff, _EB)], srcbuf.at[par], sem_e)

    def wait_edge(par):
        pltpu.make_async_copy(dst_hbm.at[pl.ds(0, _EB)], dstbuf.at[par],
                              sem_e).wait()
        pltpu.make_async_copy(src_hbm.at[pl.ds(0, _EB)], srcbuf.at[par],
                              sem_e).wait()

    def fire_g(kb, stg):
        offb = pl.multiple_of(kb * 32, 8)
        pltpu.async_copy(x_hbm.at[msrc.at[pl.ds(offb, 32)]], stg, sem_g)

    def wait_g(stg):
        pltpu.make_async_copy(x_hbm.at[pl.ds(0, 32)], stg, sem_g).wait()

    def accum(stg, kb):
        base = kb * 32

        def rowbody(r, _):
            r16 = (r // 16) * 16
            dvec = mdst[pl.ds(base + r16, 16)]
            lsel = jnp.full((16,), r, jnp.int32) & 15
            dsplat = dvec.at[lsel].get(mode="promise_in_bounds")
            for c in range(NCD):
                v = stg[r, pl.ds(16 * c, 16)]
                plsc.addupdate_scatter(acc, [dsplat, cols[c]], v)
            return 0

        lax.fori_loop(0, 32, rowbody, 0)

    def flush_from(d0, d1):
        @pl.when(d1 > d0)
        def _():
            fire_g(2 * d0, stg0)

            def fpair(jp, _):
                k0 = 2 * d0 + 2 * jp
                fire_g(k0 + 1, stg1)
                wait_g(stg0)
                accum(stg0, k0)

                @pl.when(jp < d1 - d0 - 1)
                def _():
                    fire_g(k0 + 2, stg0)

                wait_g(stg1)
                accum(stg1, k0 + 1)
                return 0

            lax.fori_loop(0, d1 - d0, fpair, 0)

    for p in range(2):
        chunk = wid * 2 + p
        lo = chunk * _CH

        def zbody(rr, _):
            rrf = jnp.full((16,), rr, jnp.int32)
            for c in range(NCD):
                plsc.store_scatter(acc, [rrf, cols[c]], zero16)
            return 0

        lax.fori_loop(0, _CH + 1, zbody, 0)

        def scanflush(par, carry):
            cntv, totv, done64, nspill = carry

            def scanbody(i, cv):
                cnv, ttv = cv
                d = dstbuf[par, pl.ds(16 * i, 16)]
                s = srcbuf[par, pl.ds(16 * i, 16)]
                m = (d >= lo) & (d < lo + _CH)
                pos = cnv + _prefix16(m, lanes) - 1
                plsc.store_scatter(msrc, [pos], s, mask=m)
                plsc.store_scatter(mdst, [pos], d - lo, mask=m)
                pc = plsc.all_reduce_population_count(m)
                return (cnv + pc, ttv + pc)

            cntv, totv = lax.fori_loop(0, _EB // 16, scanbody, (cntv, totv))
            cnt = _splat_to_scalar(cntv, 12)
            new64 = cnt // 64
            flush_from(done64, new64)

            do_spill = cnt >= 2048

            @pl.when(do_spill)
            def _():
                woff = pl.multiple_of(chunk * _STRIDE + nspill * 2048, 8)
                pltpu.async_copy(msrc.at[pl.ds(0, 2048)],
                                 lsrc_hbm.at[pl.ds(woff, 2048)],
                                 sem_s).wait()
                pltpu.async_copy(mdst.at[pl.ds(0, 2048)],
                                 ldst_hbm.at[pl.ds(woff, 2048)],
                                 sem_s).wait()

                def shbody(t, _):
                    msrc[pl.ds(16 * t, 16)] = msrc[pl.ds(2048 + 16 * t, 16)]
                    mdst[pl.ds(16 * t, 16)] = mdst[pl.ds(2048 + 16 * t, 16)]
                    return 0

                lax.fori_loop(0, (cnt - 2048 + 15) // 16, shbody, 0)

            spl = do_spill.astype(jnp.int32)
            cnt = cnt - 2048 * spl
            done64 = new64 - 32 * spl
            nspill = nspill + spl
            return (jnp.full((16,), cnt, jnp.int32), totv, done64, nspill)

        fire_edge(0, 0)

        def pairbody(j, carry):
            fire_edge(2 * j + 1, 1)
            wait_edge(0)
            carry = scanflush(0, carry)

            @pl.when(j < _NBLK // 2 - 1)
            def _():
                fire_edge(2 * j + 2, 0)

            wait_edge(1)
            carry = scanflush(1, carry)
            return carry

        z16 = jnp.zeros((16,), jnp.int32)
        carry = lax.fori_loop(0, _NBLK // 2, pairbody,
                              (z16, z16, jnp.int32(0), jnp.int32(0)))
        cntv, totv, done64, nspill = carry
        cnt = _splat_to_scalar(cntv, 12)

        for j in range(4):
            pos = cnt + 16 * j + lanes
            plsc.store_scatter(msrc, [pos], jnp.zeros((16,), jnp.int32))
            plsc.store_scatter(mdst, [pos], jnp.full((16,), _CH, jnp.int32))
        nb2 = (cnt + 63) // 64
        flush_from(done64, nb2)

        woff = pl.multiple_of(chunk * _STRIDE + nspill * 2048, 8)
        pltpu.async_copy(msrc.at[pl.ds(0, 2112)],
                         lsrc_hbm.at[pl.ds(woff, 2112)], sem_s).wait()
        pltpu.async_copy(mdst.at[pl.ds(0, 2112)],
                         ldst_hbm.at[pl.ds(woff, 2112)], sem_s).wait()

        tot = _splat_to_scalar(totv, 18)
        cbuf[pl.ds(0, 16)] = jnp.full((16,), tot, jnp.int32)
        coff = pl.multiple_of(16 * chunk, 8)
        pltpu.async_copy(cbuf, cnts_hbm.at[pl.ds(coff, 16)], sem_s).wait()

        pltpu.async_copy(acc.at[pl.ds(0, _CH)], out_hbm.at[pl.ds(lo, _CH)],
                         sem_g).wait()


def _sc_cons_body(Din, x_hbm, lsrc_hbm, ldst_hbm, cnts_hbm, out_hbm,
                  acc, stg0, stg1, lsrc, ldst, cbuf, sem_e, sem_g):
    NCD = Din // 16
    wid = lax.axis_index("s") * 2 + lax.axis_index("c")
    lanes = lax.broadcasted_iota(jnp.int32, (16,), 0)
    zero16 = jnp.zeros((16,), jnp.float32)
    cols = [lanes + 16 * c for c in range(NCD)]

    def fire_g(kb, stg):
        offb = pl.multiple_of(kb * 32, 8)
        pltpu.async_copy(x_hbm.at[lsrc.at[pl.ds(offb, 32)]], stg, sem_g)

    def wait_g(stg):
        pltpu.make_async_copy(x_hbm.at[pl.ds(0, 32)], stg, sem_g).wait()

    def accum(stg, kb):
        base = kb * 32

        def rowbody(r, _):
            r16 = (r // 16) * 16
            dvec = ldst[pl.ds(base + r16, 16)]
            lsel = jnp.full((16,), r, jnp.int32) & 15
            dsplat = dvec.at[lsel].get(mode="promise_in_bounds")
            for c in range(NCD):
                v = stg[r, pl.ds(16 * c, 16)]
                plsc.addupdate_scatter(acc, [dsplat, cols[c]], v)
            return 0

        lax.fori_loop(0, 32, rowbody, 0)

    def flush_from(d0, d1):
        @pl.when(d1 > d0)
        def _():
            fire_g(2 * d0, stg0)

            def fpair(jp, _):
                k0 = 2 * d0 + 2 * jp
                fire_g(k0 + 1, stg1)
                wait_g(stg0)
                accum(stg0, k0)

                @pl.when(jp < d1 - d0 - 1)
                def _():
                    fire_g(k0 + 2, stg0)

                wait_g(stg1)
                accum(stg1, k0 + 1)
                return 0

            lax.fori_loop(0, d1 - d0, fpair, 0)

    for p in range(2):
        chunk = wid * 2 + p
        lo = chunk * _CH

        def zbody(rr, _):
            rrf = jnp.full((16,), rr, jnp.int32)
            for c in range(NCD):
                plsc.store_scatter(acc, [rrf, cols[c]], zero16)
            return 0

        lax.fori_loop(0, _CH + 1, zbody, 0)

        coff = pl.multiple_of(16 * chunk, 8)
        pltpu.async_copy(cnts_hbm.at[pl.ds(coff, 16)], cbuf, sem_e).wait()
        cnt = _splat_to_scalar(cbuf[pl.ds(0, 16)], 18)
        nb64 = (cnt + 63) // 64
        nblk = (nb64 + 31) // 32

        def blkb(b, _):
            boff = pl.multiple_of(chunk * _STRIDE + b * 2048, 8)
            pltpu.async_copy(lsrc_hbm.at[pl.ds(boff, 2048)], lsrc, sem_e)
            pltpu.async_copy(ldst_hbm.at[pl.ds(boff, 2048)], ldst, sem_e)
            pltpu.make_async_copy(lsrc_hbm.at[pl.ds(0, 2048)], lsrc,
                                  sem_e).wait()
            pltpu.make_async_copy(ldst_hbm.at[pl.ds(0, 2048)], ldst,
                                  sem_e).wait()
            kn = jnp.minimum(nb64 - b * 32, 32)
            flush_from(0, kn)
            return 0

        lax.fori_loop(0, nblk, blkb, 0)

        pltpu.async_copy(acc.at[pl.ds(0, _CH)], out_hbm.at[pl.ds(lo, _CH)],
                         sem_g).wait()


_SC_PARAMS = dict(use_tc_tiling_on_sc=False, needs_layout_passes=False)


@functools.lru_cache(maxsize=None)
def _make_sc_build():
    mesh = plsc.VectorSubcoreMesh(core_axis_name="c", subcore_axis_name="s")
    return pl.kernel(
        _sc_build_body,
        out_type=(jax.ShapeDtypeStruct((_NPAD, 256), jnp.float32),
                  jax.ShapeDtypeStruct((64 * _STRIDE,), jnp.int32),
                  jax.ShapeDtypeStruct((64 * _STRIDE,), jnp.int32),
                  jax.ShapeDtypeStruct((1024,), jnp.int32)),
        mesh=mesh,
        compiler_params=pltpu.CompilerParams(**_SC_PARAMS),
        scratch_types=[
            pltpu.VMEM((_CH + 1, 256), jnp.float32),
            pltpu.VMEM((32, 256), jnp.float32),
            pltpu.VMEM((32, 256), jnp.float32),
            pltpu.VMEM((2, _EB), jnp.int32),
            pltpu.VMEM((2, _EB), jnp.int32),
            pltpu.VMEM((_MBB,), jnp.int32),
            pltpu.VMEM((_MBB,), jnp.int32),
            pltpu.VMEM((16,), jnp.int32),
            pltpu.SemaphoreType.DMA,
            pltpu.SemaphoreType.DMA,
            pltpu.SemaphoreType.DMA,
        ],
    )


@functools.lru_cache(maxsize=None)
def _make_sc_cons(Din):
    mesh = plsc.VectorSubcoreMesh(core_axis_name="c", subcore_axis_name="s")
    return pl.kernel(
        functools.partial(_sc_cons_body, Din),
        out_type=jax.ShapeDtypeStruct((_NPAD, Din), jnp.float32),
        mesh=mesh,
        compiler_params=pltpu.CompilerParams(**_SC_PARAMS),
        scratch_types=[
            pltpu.VMEM((_CH + 1, Din), jnp.float32),
            pltpu.VMEM((32, Din), jnp.float32),
            pltpu.VMEM((32, Din), jnp.float32),
            pltpu.VMEM((2048,), jnp.int32),
            pltpu.VMEM((2048,), jnp.int32),
            pltpu.VMEM((16,), jnp.int32),
            pltpu.SemaphoreType.DMA,
            pltpu.SemaphoreType.DMA,
        ],
    )


def _mlpA_body(x_ref, agg_ref, wa_ref, ba_ref, o_ref):
    h = x_ref[...] + agg_ref[...]
    y = jnp.dot(h, wa_ref[...], preferred_element_type=jnp.float32)
    o_ref[...] = jnp.maximum(y + ba_ref[...], 0.0)


def _mlpB_body(h_ref, wb_ref, bb_ref, r_ref, s_ref, q_ref):
    y = jnp.dot(h_ref[...], wb_ref[...], preferred_element_type=jnp.float32)
    r = jnp.maximum(y + bb_ref[...], 0.0)
    r_ref[...] = r

    @pl.when(pl.program_id(0) == 0)
    def _():
        s_ref[...] = jnp.zeros_like(s_ref)
        q_ref[...] = jnp.zeros_like(q_ref)

    s_ref[...] += jnp.sum(r, axis=0, keepdims=True)
    q_ref[...] += jnp.sum(r * r, axis=0, keepdims=True)


def _bn_body(r_ref, s_ref, q_ref, g_ref, be_ref, o_ref):
    mu = s_ref[...] * (1.0 / N)
    var = q_ref[...] * (1.0 / N) - mu * mu
    inv = lax.rsqrt(var + _EPS)
    o_ref[...] = g_ref[...] * ((r_ref[...] - mu) * inv) + be_ref[...]


def _pool_head_body(r_ref, s_ref, q_ref, g_ref, be_ref, ids_ref,
                    w1_ref, b1_ref, w2_ref, b2_ref, o_ref, pool_ref):
    i = pl.program_id(0)
    mu = s_ref[...] * (1.0 / N)
    var = q_ref[...] * (1.0 / N) - mu * mu
    inv = lax.rsqrt(var + _EPS)
    xn = g_ref[...] * ((r_ref[...] - mu) * inv) + be_ref[...]

    @pl.when(i == 0)
    def _():
        pool_ref[...] = jnp.full_like(pool_ref, -jnp.inf)

    idsb = ids_ref[...]
    for g in range(G):
        mask = idsb == g

        @pl.when(jnp.any(mask))
        def _():
            for cc in range(H // 128):
                sel = jnp.where(mask, xn[:, cc * 128:(cc + 1) * 128],
                                -jnp.inf)
                m = jnp.max(sel, axis=0)
                cur = pool_ref[g, pl.ds(cc * 128, 128)]
                pool_ref[g, pl.ds(cc * 128, 128)] = jnp.maximum(cur, m)

    @pl.when(i == NT - 1)
    def _():
        p = pool_ref[...]
        h = jnp.maximum(
            jnp.dot(p, w1_ref[...], preferred_element_type=jnp.float32)
            + b1_ref[...], 0.0)
        o_ref[...] = (
            jnp.dot(h, w2_ref[...], preferred_element_type=jnp.float32)
            + b2_ref[...])


def _row_spec(d):
    return pl.BlockSpec((R, d), lambda i: (i, 0))


def _full_spec(shape):
    nd = len(shape)
    return pl.BlockSpec(shape, lambda i: (0,) * nd)


def _layer(x, agg, Wa, ba, Wb, bb):
    d = x.shape[1]
    h1 = pl.pallas_call(
        _mlpA_body,
        grid=(NT,),
        in_specs=[_row_spec(d), _row_spec(d),
                  _full_spec((d, H)), _full_spec((1, H))],
        out_specs=_row_spec(H),
        out_shape=jax.ShapeDtypeStruct((N, H), jnp.float32),
    )(x, agg, Wa, ba[None])
    r, s, q = pl.pallas_call(
        _mlpB_body,
        grid=(NT,),
        in_specs=[_row_spec(H), _full_spec((H, H)), _full_spec((1, H))],
        out_specs=[_row_spec(H), _full_spec((1, H)), _full_spec((1, H))],
        out_shape=[jax.ShapeDtypeStruct((N, H), jnp.float32),
                   jax.ShapeDtypeStruct((1, H), jnp.float32),
                   jax.ShapeDtypeStruct((1, H), jnp.float32)],
    )(h1, Wb, bb[None])
    return r, s, q


def _bn(r, s, q, g, be):
    return pl.pallas_call(
        _bn_body,
        grid=(NT,),
        in_specs=[_row_spec(H), _full_spec((1, H)), _full_spec((1, H)),
                  _full_spec((1, H)), _full_spec((1, H))],
        out_specs=_row_spec(H),
        out_shape=jax.ShapeDtypeStruct((N, H), jnp.float32),
    )(r, s[None] if s.ndim == 1 else s, q, g[None], be[None])


def _pool_head(r, s, q, g, be, batch, Wf1, bf1, Wf2, bf2):
    C = Wf1.shape[1]
    P = 128
    w1 = jnp.zeros((H, P), jnp.float32).at[:, :C].set(Wf1)
    b1 = jnp.zeros((1, P), jnp.float32).at[0, :C].set(bf1)
    w2 = jnp.zeros((P, P), jnp.float32).at[:C, :C].set(Wf2)
    b2 = jnp.zeros((1, P), jnp.float32).at[0, :C].set(bf2)
    ids = jnp.broadcast_to(batch[:, None], (N, 128))
    out = pl.pallas_call(
        _pool_head_body,
        grid=(NT,),
        in_specs=[_row_spec(H), _full_spec((1, H)), _full_spec((1, H)),
                  _full_spec((1, H)), _full_spec((1, H)),
                  pl.BlockSpec((R, 128), lambda i: (i, 0)),
                  _full_spec((H, P)), _full_spec((1, P)),
                  _full_spec((P, P)), _full_spec((1, P))],
        out_specs=_full_spec((G, P)),
        out_shape=jax.ShapeDtypeStruct((G, P), jnp.float32),
        scratch_shapes=[pltpu.VMEM((G, H), jnp.float32)],
    )(r, s, q, g[None], be[None], ids, w1, b1, w2, b2)
    return out[:, :C]


@jax.jit
def kernel(data_base, edge_index_base, batch_base,
           W1a, b1a, W1b, b1b, g1, be1,
           W2a, b2a, W2b, b2b, g2, be2,
           W3a, b3a, W3b, b3b, g3, be3,
           Wf1, bf1, Wf2, bf2):
    src = edge_index_base[0]
    dst = edge_index_base[1]

    x0 = data_base
    agg1, ls, ld, cns = _make_sc_build()(x0, src, dst)
    r1, s1, q1 = _layer(x0, agg1[:N], W1a, b1a, W1b, b1b)
    x1 = _bn(r1, s1, q1, g1, be1)
    agg2 = _make_sc_cons(512)(x1, ls, ld, cns)[:N]
    r2, s2, q2 = _layer(x1, agg2, W2a, b2a, W2b, b2b)
    x2 = _bn(r2, s2, q2, g2, be2)
    agg3 = _make_sc_cons(512)(x2, ls, ld, cns)[:N]
    r3, s3, q3 = _layer(x2, agg3, W3a, b3a, W3b, b3b)
    return _pool_head(r3, s3, q3, g3, be3, batch_base, Wf1, bf1, Wf2, bf2)
